# baseline (speedup 1.0000x reference)
import functools

import jax
import jax.numpy as jnp
from jax import lax
from jax.experimental import pallas as pl
from jax.experimental.pallas import tpu as pltpu
from jax.experimental.pallas import tpu_sc as plsc

SEQ_LEN = 1024
EMBED_DIM = 16
VOCAB = 128

_info = plsc.get_sparse_core_info()
_NC, _NS, _L = _info.num_cores, _info.num_subcores, _info.num_lanes
_NW = _NC * _NS
_BPW = SEQ_LEN // _NW
_NGROUPS = _BPW // _L

_mesh = plsc.VectorSubcoreMesh(core_axis_name="c", subcore_axis_name="s")


@functools.partial(
    pl.kernel,
    mesh=_mesh,
    out_type=jax.ShapeDtypeStruct((SEQ_LEN, EMBED_DIM), jnp.float32),
    compiler_params=pltpu.CompilerParams(needs_layout_passes=False),
    scratch_types=[
        pltpu.VMEM((_BPW,), jnp.int32),
        pltpu.VMEM((VOCAB, EMBED_DIM), jnp.float32),
        pltpu.VMEM((_BPW, EMBED_DIM), jnp.float32),
        pltpu.SemaphoreType.DMA,
        pltpu.SemaphoreType.DMA,
        pltpu.SemaphoreType.DMA,
    ],
)
def _embed_add(idx_hbm, tok_hbm, pos_hbm, out_hbm, idx_v, tok_v, out_v,
               idx_sem, tok_sem, pos_sem):
    wid = lax.axis_index("s") * _NC + lax.axis_index("c")
    base = wid * _BPW
    idx_cp = pltpu.async_copy(idx_hbm.at[pl.ds(base, _BPW)], idx_v, idx_sem)
    tok_cp = pltpu.async_copy(tok_hbm, tok_v, tok_sem)
    pos_cp = pltpu.async_copy(pos_hbm.at[pl.ds(base, _BPW)], out_v, pos_sem)
    idx_cp.wait()
    tok_cp.wait()
    pos_cp.wait()
    for g in range(_NGROUPS):
        tok_idx = idx_v[pl.ds(g * _L, _L)]
        rows = lax.iota(jnp.int32, _L) + (g * _L)
        for d in range(EMBED_DIM):
            col = jnp.full((_L,), d, jnp.int32)
            vals = plsc.load_gather(tok_v, [tok_idx, col])
            plsc.addupdate_scatter(out_v, [rows, col], vals)
    pltpu.sync_copy(out_v, out_hbm.at[pl.ds(base, _BPW)])


def kernel(encoded, token_table, pos_table):
    return _embed_add(encoded.astype(jnp.int32), token_table, pos_table)


# speedup vs baseline: 1.1877x; 1.1877x over previous
import functools

import jax
import jax.numpy as jnp
from jax import lax
from jax.experimental import pallas as pl
from jax.experimental.pallas import tpu as pltpu
from jax.experimental.pallas import tpu_sc as plsc

SEQ_LEN = 1024
EMBED_DIM = 16
VOCAB = 128
BLOCK = 128
_NBLOCKS = SEQ_LEN // BLOCK

_info = plsc.get_sparse_core_info()
_NC, _NS, _L = _info.num_cores, _info.num_subcores, _info.num_lanes

_mesh = plsc.VectorSubcoreMesh(core_axis_name="c", subcore_axis_name="s")


@functools.partial(
    pl.kernel,
    mesh=_mesh,
    out_type=jax.ShapeDtypeStruct((EMBED_DIM, SEQ_LEN), jnp.float32),
    compiler_params=pltpu.CompilerParams(needs_layout_passes=False),
    scratch_types=[
        pltpu.VMEM((BLOCK,), jnp.int32),
        pltpu.VMEM((EMBED_DIM, VOCAB), jnp.float32),
        pltpu.VMEM((EMBED_DIM, BLOCK), jnp.float32),
        pltpu.VMEM((EMBED_DIM, BLOCK), jnp.float32),
        pltpu.SemaphoreType.DMA,
        pltpu.SemaphoreType.DMA,
        pltpu.SemaphoreType.DMA,
    ],
)
def _embed_add(idx_hbm, tok_hbm, pos_hbm, out_hbm, idx_v, tok_v, pos_v,
               out_v, idx_sem, tok_sem, pos_sem):
    wid = lax.axis_index("s") * _NC + lax.axis_index("c")

    @pl.when(wid < _NBLOCKS)
    def _():
        base = wid * BLOCK
        idx_cp = pltpu.async_copy(idx_hbm.at[pl.ds(base, BLOCK)], idx_v,
                                  idx_sem)
        tok_cp = pltpu.async_copy(tok_hbm, tok_v, tok_sem)
        pos_cp = pltpu.async_copy(pos_hbm.at[:, pl.ds(base, BLOCK)], pos_v,
                                  pos_sem)
        idx_cp.wait()
        tok_cp.wait()
        pos_cp.wait()
        for g in range(BLOCK // _L):
            tok_idx = idx_v[pl.ds(g * _L, _L)]
            for d in range(EMBED_DIM):
                dvec = jnp.full((_L,), d, jnp.int32)
                vals = plsc.load_gather(tok_v, [dvec, tok_idx])
                out_v[d, pl.ds(g * _L, _L)] = (
                    vals + pos_v[d, pl.ds(g * _L, _L)])
        pltpu.sync_copy(out_v, out_hbm.at[:, pl.ds(base, BLOCK)])


def kernel(encoded, token_table, pos_table):
    out_t = _embed_add(encoded.astype(jnp.int32), token_table.T, pos_table.T)
    return out_t.T


# speedup vs baseline: 1.2318x; 1.0372x over previous
import functools

import jax
import jax.numpy as jnp
from jax import lax
from jax.experimental import pallas as pl
from jax.experimental.pallas import tpu as pltpu
from jax.experimental.pallas import tpu_sc as plsc

SEQ_LEN = 1024
EMBED_DIM = 16
VOCAB = 128
BLOCK = 128
_NBLOCKS = SEQ_LEN // BLOCK

_info = plsc.get_sparse_core_info()
_NC, _NS, _L = _info.num_cores, _info.num_subcores, _info.num_lanes

_mesh = plsc.VectorSubcoreMesh(core_axis_name="c", subcore_axis_name="s")


@functools.partial(
    pl.kernel,
    mesh=_mesh,
    out_type=jax.ShapeDtypeStruct((EMBED_DIM, SEQ_LEN), jnp.float32),
    compiler_params=pltpu.CompilerParams(needs_layout_passes=False),
    scratch_types=[
        pltpu.VMEM((BLOCK,), jnp.int32),
        pltpu.VMEM((EMBED_DIM, VOCAB), jnp.float32),
        pltpu.VMEM((EMBED_DIM, BLOCK), jnp.float32),
        pltpu.VMEM((EMBED_DIM, BLOCK), jnp.float32),
        pltpu.SemaphoreType.DMA,
        pltpu.SemaphoreType.DMA,
        pltpu.SemaphoreType.DMA,
    ],
)
def _embed_add(idx_hbm, tok_hbm, pos_hbm, out_hbm, idx_v, tok_v, pos_v,
               out_v, idx_sem, tok_sem, pos_sem):
    wid = lax.axis_index("s") * _NC + lax.axis_index("c")

    @pl.when(wid < _NBLOCKS)
    def _():
        base = wid * BLOCK
        idx_cp = pltpu.async_copy(idx_hbm.at[pl.ds(base, BLOCK)], idx_v,
                                  idx_sem)
        tok_cp = pltpu.async_copy(tok_hbm, tok_v, tok_sem)
        pos_cp = pltpu.async_copy(pos_hbm.at[:, pl.ds(base, BLOCK)], pos_v,
                                  pos_sem)
        idx_cp.wait()
        tok_cp.wait()
        pos_cp.wait()
        def body(g, carry):
            off = g * _L
            tok_idx = idx_v[pl.ds(off, _L)]
            for d in range(EMBED_DIM):
                dvec = jnp.full((_L,), d, jnp.int32)
                vals = plsc.load_gather(tok_v, [dvec, tok_idx])
                out_v[d, pl.ds(off, _L)] = vals + pos_v[d, pl.ds(off, _L)]
            return carry

        lax.fori_loop(0, BLOCK // _L, body, 0)
        pltpu.sync_copy(out_v, out_hbm.at[:, pl.ds(base, BLOCK)])


def kernel(encoded, token_table, pos_table):
    out_t = _embed_add(encoded.astype(jnp.int32), token_table.T, pos_table.T)
    return out_t.T
